# baseline (speedup 1.0000x reference)
import functools

import jax
import jax.numpy as jnp
from jax import lax
from jax.experimental import pallas as pl
from jax.experimental.pallas import tpu as pltpu
from jax.experimental.pallas import tpu_sc as plsc

EMBED_DIM = 64
NUM_EMB_ROWS = 144
NUM_CORES = 2
NUM_SUBCORES = 16
NUM_WORKERS = NUM_CORES * NUM_SUBCORES

CHUNK = 512
IDX_PER_STREAM = 512
NBUF = 2
BLK = 2048
SC_ROWS = 30 * NUM_WORKERS * CHUNK * NBUF // 2


def _run_sc(idx_flat, table):
    n = idx_flat.shape[0]
    per_w = n // NUM_WORKERS
    steps = per_w // CHUNK

    mesh = plsc.VectorSubcoreMesh(
        core_axis_name="c", subcore_axis_name="s",
        num_cores=NUM_CORES, num_subcores=NUM_SUBCORES)

    @functools.partial(
        pl.kernel,
        out_type=jax.ShapeDtypeStruct((n, EMBED_DIM), jnp.float32),
        mesh=mesh,
        scratch_types=[
            pltpu.VMEM((NBUF, CHUNK), jnp.int32),
            pltpu.VMEM((NBUF, CHUNK, EMBED_DIM), jnp.float32),
            pltpu.VMEM_SHARED((NUM_EMB_ROWS, EMBED_DIM), jnp.float32),
            [pltpu.SemaphoreType.DMA] * NBUF,
            [pltpu.SemaphoreType.DMA] * NBUF,
            pltpu.SemaphoreType.DMA,
        ],
        compiler_params=pltpu.CompilerParams(use_tc_tiling_on_sc=False),
    )
    def k(idx_hbm, table_hbm, out_hbm, idx_v, rows_v, table_v,
          sem_idx, sem_out, sem_g):
        sid = lax.axis_index("s")
        wid = sid * NUM_CORES + lax.axis_index("c")
        base = wid * per_w

        @pl.when(sid == 0)
        def _stage_table():
            pltpu.sync_copy(table_hbm, table_v)

        plsc.subcore_barrier()

        def idx_copy(step, b):
            return pltpu.make_async_copy(
                idx_hbm.at[pl.ds(base + step * CHUNK, CHUNK)],
                idx_v.at[b], sem_idx[b])

        def out_copy(step, b):
            return pltpu.make_async_copy(
                rows_v.at[b],
                out_hbm.at[pl.ds(base + step * CHUNK, CHUNK)],
                sem_out[b])

        for b in range(NBUF):
            idx_copy(b, b).start()

        def body(g, _):
            for b in range(NBUF):
                step = g * NBUF + b
                idx_copy(step, b).wait()
                @pl.when(g > 0)
                def _drain_prev():
                    out_copy(step - NBUF, b).wait()
                copies = []
                for j in range(CHUNK // IDX_PER_STREAM):
                    sl = pl.ds(j * IDX_PER_STREAM, IDX_PER_STREAM)
                    copies.append(pltpu.async_copy(
                        table_v.at[idx_v.at[b].at[sl]],
                        rows_v.at[b].at[sl], sem_g))
                for c in copies:
                    c.wait()
                @pl.when(step + NBUF < steps)
                def _prefetch():
                    idx_copy(step + NBUF, b).start()
                out_copy(step, b).start()
            return _

        lax.fori_loop(0, steps // NBUF, body, 0)
        for b in range(NBUF):
            out_copy(steps - NBUF + b, b).wait()

    return k(idx_flat, table)


def _tc_body(idx_ref, tab_ref, out_ref):
    idxr = idx_ref[0]
    iot = lax.broadcasted_iota(jnp.int32, (NUM_EMB_ROWS, BLK), 0)
    oht = jnp.where(iot == idxr, 1.0, 0.0).astype(jnp.float32)
    out_ref[0] = lax.dot_general(
        oht, tab_ref[...],
        dimension_numbers=(((0,), (0,)), ((), ())),
        preferred_element_type=jnp.float32)


def _run_tc(idx_flat, table):
    n = idx_flat.shape[0]
    nb = n // BLK
    idx3 = idx_flat.reshape(nb, 1, BLK)
    out = pl.pallas_call(
        _tc_body,
        grid=(nb,),
        in_specs=[
            pl.BlockSpec((1, 1, BLK), lambda i: (i, 0, 0)),
            pl.BlockSpec((NUM_EMB_ROWS, EMBED_DIM), lambda i: (0, 0)),
        ],
        out_specs=pl.BlockSpec((1, BLK, EMBED_DIM), lambda i: (i, 0, 0)),
        out_shape=jax.ShapeDtypeStruct((nb, BLK, EMBED_DIM), jnp.float32),
    )(idx3, table)
    return out.reshape(n, EMBED_DIM)


def kernel(channel_indices, table):
    b, f = channel_indices.shape
    idx_flat = channel_indices.reshape(b * f).astype(jnp.int32)
    out_sc = _run_sc(idx_flat[:SC_ROWS], table)
    out_tc = _run_tc(idx_flat[SC_ROWS:], table)
    out = jnp.concatenate([out_sc, out_tc], axis=0)
    return out.reshape(b, f, EMBED_DIM)


# speedup vs baseline: 8.8250x; 8.8250x over previous
import functools

import jax
import jax.numpy as jnp
from jax import lax
from jax.experimental import pallas as pl
from jax.experimental.pallas import tpu as pltpu
from jax.experimental.pallas import tpu_sc as plsc

EMBED_DIM = 64
NUM_CORES = 2
NUM_SUBCORES = 16
NUM_WORKERS = NUM_CORES * NUM_SUBCORES

CHUNK = 512
IDX_PER_STREAM = 512
NBUF = 2


@jax.jit
def _run(idx_flat, table):
    n = idx_flat.shape[0]
    per_w = n // NUM_WORKERS
    steps = per_w // CHUNK
    assert steps % NBUF == 0

    mesh = plsc.VectorSubcoreMesh(
        core_axis_name="c", subcore_axis_name="s",
        num_cores=NUM_CORES, num_subcores=NUM_SUBCORES)

    @functools.partial(
        pl.kernel,
        out_type=jax.ShapeDtypeStruct((n, EMBED_DIM), jnp.float32),
        mesh=mesh,
        scratch_types=[
            pltpu.VMEM((NBUF, CHUNK), jnp.int32),
            pltpu.VMEM((NBUF, CHUNK, EMBED_DIM), jnp.float32),
            pltpu.VMEM_SHARED((144, EMBED_DIM), jnp.float32),
            [pltpu.SemaphoreType.DMA] * NBUF,
            [pltpu.SemaphoreType.DMA] * NBUF,
            pltpu.SemaphoreType.DMA,
        ],
        compiler_params=pltpu.CompilerParams(use_tc_tiling_on_sc=False),
    )
    def k(idx_hbm, table_hbm, out_hbm, idx_v, rows_v, table_v,
          sem_idx, sem_out, sem_g):
        sid = lax.axis_index("s")
        wid = sid * NUM_CORES + lax.axis_index("c")
        base = wid * per_w

        @pl.when(sid == 0)
        def _stage_table():
            pltpu.sync_copy(table_hbm, table_v)

        plsc.subcore_barrier()

        def idx_copy(step, b):
            return pltpu.make_async_copy(
                idx_hbm.at[pl.ds(base + step * CHUNK, CHUNK)],
                idx_v.at[b], sem_idx[b])

        def out_copy(step, b):
            return pltpu.make_async_copy(
                rows_v.at[b],
                out_hbm.at[pl.ds(base + step * CHUNK, CHUNK)],
                sem_out[b])

        for b in range(NBUF):
            idx_copy(b, b).start()

        def body(g, _):
            for b in range(NBUF):
                step = g * NBUF + b
                idx_copy(step, b).wait()
                @pl.when(g > 0)
                def _drain_prev():
                    out_copy(step - NBUF, b).wait()
                copies = []
                for j in range(CHUNK // IDX_PER_STREAM):
                    sl = pl.ds(j * IDX_PER_STREAM, IDX_PER_STREAM)
                    copies.append(pltpu.async_copy(
                        table_v.at[idx_v.at[b].at[sl]],
                        rows_v.at[b].at[sl], sem_g))
                for c in copies:
                    c.wait()
                @pl.when(step + NBUF < steps)
                def _prefetch():
                    idx_copy(step + NBUF, b).start()
                out_copy(step, b).start()
            return _

        lax.fori_loop(0, steps // NBUF, body, 0)
        for b in range(NBUF):
            out_copy(steps - NBUF + b, b).wait()

    return k(idx_flat, table)


def kernel(channel_indices, table):
    b, f = channel_indices.shape
    idx_flat = channel_indices.reshape(b * f).astype(jnp.int32)
    out = _run(idx_flat, table)
    return out.reshape(b, f, EMBED_DIM)
